# baseline (speedup 1.0000x reference)
import functools

import jax
import jax.numpy as jnp
from jax import lax
from jax.experimental import pallas as pl
from jax.experimental.pallas import tpu as pltpu
from jax.experimental.pallas import tpu_sc as plsc

BATCH = 4096
T = 200
D = 64
LANES = 16

NC = 2
NS = 16
NW = NC * NS

FLAT = BATCH * T
ROWS_PER_W = FLAT // NW
SUB = 128
UNIT = 256
NUNITS = ROWS_PER_W // UNIT
KU = UNIT // SUB
BLKS = BATCH // UNIT

_mesh = plsc.VectorSubcoreMesh(core_axis_name="c", subcore_axis_name="s")


@functools.partial(
    pl.kernel,
    out_type=jax.ShapeDtypeStruct((T, D // 8, BATCH // SUB, 8, SUB), jnp.float32),
    mesh=_mesh,
    compiler_params=pltpu.CompilerParams(
        use_tc_tiling_on_sc=False, needs_layout_passes=False
    ),
    scratch_types=[
        pltpu.VMEM((ROWS_PER_W // SUB, SUB), jnp.int32),
        pltpu.VMEM((2, UNIT, D), jnp.float32),
        pltpu.VMEM((2, 8, KU, 8, SUB), jnp.float32),
        pltpu.VMEM((T, D), jnp.float32),
        pltpu.SemaphoreType.DMA,
        pltpu.SemaphoreType.DMA,
        pltpu.SemaphoreType.DMA,
        pltpu.SemaphoreType.DMA,
    ],
)
def _emb_lookup(x_hbm, tab_hbm, pemb_hbm, out_hbm, idx_v, rows_v, out_v,
                pemb_v, gsem0, gsem1, ssem0, ssem1):
    wid = lax.axis_index("s") * NC + lax.axis_index("c")
    gsems = (gsem0, gsem1)
    ssems = (ssem0, ssem1)
    pltpu.sync_copy(pemb_hbm, pemb_v)
    pltpu.sync_copy(x_hbm.at[wid], idx_v)

    io = lax.iota(jnp.int32, 16)

    def fire_gathers(n, s):
        for j in range(KU):
            pltpu.make_async_copy(
                tab_hbm.at[idx_v.at[KU * n + j]],
                rows_v.at[s, pl.ds(j * SUB, SUB)],
                gsems[s],
            ).start()

    def drain_gathers(n, s):
        for j in range(KU):
            pltpu.make_async_copy(
                tab_hbm.at[idx_v.at[KU * n + j]],
                rows_v.at[s, pl.ds(j * SUB, SUB)],
                gsems[s],
            ).wait()

    def scatter_desc(n, s):
        u = wid * NUNITS + n
        t = u // BLKS
        blk = u % BLKS
        return pltpu.make_async_copy(
            out_v.at[s],
            out_hbm.at[t, :, pl.ds(blk * KU, KU)],
            ssems[s],
        )

    fire_gathers(0, 0)

    def do_pair(p, carry):
        for s in range(2):
            n = 2 * p + s
            u = wid * NUNITS + n
            t = u // BLKS

            @pl.when(n + 1 < NUNITS)
            def _():
                fire_gathers(n + 1, 1 - s)

            drain_gathers(n, s)

            pvs = [pemb_v[t, pl.ds(16 * c, 16)] for c in range(D // LANES)]

            @pl.loop(0, UNIT, unroll=2)
            def _(bb):
                for c in range(D // LANES):
                    sl = pl.ds(16 * c, 16)
                    rows_v[s, bb, sl] = rows_v[s, bb, sl] + pvs[c]

            @pl.when(n >= 2)
            def _():
                scatter_desc(n, s).wait()

            for d in range(D):
                dvec = io * 0 + d

                @pl.loop(0, UNIT // LANES, init_carry=io, unroll=4)
                def _(gi, bbvec, dvec=dvec, d=d, s=s):
                    val = plsc.load_gather(rows_v.at[s], [bbvec, dvec])
                    k2 = gi >> 3
                    col0 = (gi & 7) * 16
                    out_v[s, d >> 3, k2, d & 7, pl.ds(col0, 16)] = val
                    return bbvec + 16

            scatter_desc(n, s).start()
        return carry

    lax.fori_loop(0, NUNITS // 2, do_pair, 0)
    scatter_desc(NUNITS - 2, 0).wait()
    scatter_desc(NUNITS - 1, 1).wait()


def kernel(x, t_emb, p_emb):
    xw = x.T.astype(jnp.int32).reshape(NW, ROWS_PER_W // SUB, SUB)
    out5 = _emb_lookup(xw, t_emb, p_emb)
    return jnp.transpose(out5, (2, 4, 0, 1, 3)).reshape(BATCH, T, D)


# speedup vs baseline: 1.1728x; 1.1728x over previous
import functools

import jax
import jax.numpy as jnp
from jax import lax
from jax.experimental import pallas as pl
from jax.experimental.pallas import tpu as pltpu
from jax.experimental.pallas import tpu_sc as plsc

BATCH = 4096
T = 200
D = 64
LANES = 16

NC = 2
NS = 16
NW = NC * NS

FLAT = BATCH * T
ROWS_PER_W = FLAT // NW
SUB = 128
UNIT = 256
NUNITS = ROWS_PER_W // UNIT
KU = UNIT // SUB
BLKS = BATCH // UNIT

_mesh = plsc.VectorSubcoreMesh(core_axis_name="c", subcore_axis_name="s")


@functools.partial(
    pl.kernel,
    out_type=jax.ShapeDtypeStruct((T, D // 8, BATCH // SUB, 8, SUB), jnp.float32),
    mesh=_mesh,
    compiler_params=pltpu.CompilerParams(
        use_tc_tiling_on_sc=False, needs_layout_passes=False
    ),
    scratch_types=[
        pltpu.VMEM((ROWS_PER_W // SUB, SUB), jnp.int32),
        pltpu.VMEM((2, UNIT, D), jnp.float32),
        pltpu.VMEM((2, 8, KU, 8, SUB), jnp.float32),
        pltpu.VMEM((T, D), jnp.float32),
        pltpu.SemaphoreType.DMA,
        pltpu.SemaphoreType.DMA,
        pltpu.SemaphoreType.DMA,
        pltpu.SemaphoreType.DMA,
    ],
)
def _emb_lookup(x_hbm, tab_hbm, pemb_hbm, out_hbm, idx_v, rows_v, out_v,
                pemb_v, gsem0, gsem1, ssem0, ssem1):
    wid = lax.axis_index("s") * NC + lax.axis_index("c")
    gsems = (gsem0, gsem1)
    ssems = (ssem0, ssem1)
    pltpu.sync_copy(pemb_hbm, pemb_v)
    pltpu.sync_copy(x_hbm.at[wid], idx_v)

    io = lax.iota(jnp.int32, 16)

    def fire_gathers(n, s):
        for j in range(KU):
            pltpu.make_async_copy(
                tab_hbm.at[idx_v.at[KU * n + j]],
                rows_v.at[s, pl.ds(j * SUB, SUB)],
                gsems[s],
            ).start()

    def drain_gathers(n, s):
        for j in range(KU):
            pltpu.make_async_copy(
                tab_hbm.at[idx_v.at[KU * n + j]],
                rows_v.at[s, pl.ds(j * SUB, SUB)],
                gsems[s],
            ).wait()

    def scatter_desc(n, s):
        u = wid * NUNITS + n
        t = u // BLKS
        blk = u % BLKS
        return pltpu.make_async_copy(
            out_v.at[s],
            out_hbm.at[t, :, pl.ds(blk * KU, KU)],
            ssems[s],
        )

    fire_gathers(0, 0)

    def do_pair(p, carry):
        for s in range(2):
            n = 2 * p + s
            u = wid * NUNITS + n
            t = u // BLKS

            @pl.when(n + 1 < NUNITS)
            def _():
                fire_gathers(n + 1, 1 - s)

            drain_gathers(n, s)

            @pl.when(n >= 2)
            def _():
                scatter_desc(n, s).wait()

            tvec = lax.broadcast(t, (16,))
            for d in range(D):
                dvec = io * 0 + d
                psplat = plsc.load_gather(pemb_v, [tvec, dvec])

                @plsc.parallel_loop(0, UNIT // LANES, carry=io)
                def _(gi, bbvec, dvec=dvec, psplat=psplat, d=d, s=s):
                    val = plsc.load_gather(rows_v.at[s], [bbvec, dvec]) + psplat
                    k2 = gi >> 3
                    col0 = (gi & 7) * 16
                    out_v[s, d >> 3, k2, d & 7, pl.ds(col0, 16)] = val
                    return bbvec + 16

            scatter_desc(n, s).start()
        return carry

    lax.fori_loop(0, NUNITS // 2, do_pair, 0)
    scatter_desc(NUNITS - 2, 0).wait()
    scatter_desc(NUNITS - 1, 1).wait()


def kernel(x, t_emb, p_emb):
    xw = x.T.astype(jnp.int32).reshape(NW, ROWS_PER_W // SUB, SUB)
    out5 = _emb_lookup(xw, t_emb, p_emb)
    return jnp.transpose(out5, (2, 4, 0, 1, 3)).reshape(BATCH, T, D)


# speedup vs baseline: 2.7071x; 2.3083x over previous
import functools

import jax
import jax.numpy as jnp
from jax import lax
from jax.experimental import pallas as pl
from jax.experimental.pallas import tpu as pltpu
from jax.experimental.pallas import tpu_sc as plsc

BATCH = 4096
T = 200
D = 64
LANES = 16

NC = 2
NS = 16
NW = NC * NS

FLAT = BATCH * T
ROWS_PER_W = FLAT // NW
SUB = 128
UNIT = 256
NUNITS = ROWS_PER_W // UNIT
KU = UNIT // SUB
BLKS = BATCH // UNIT

_mesh = plsc.VectorSubcoreMesh(core_axis_name="c", subcore_axis_name="s")


@functools.partial(
    pl.kernel,
    out_type=jax.ShapeDtypeStruct((T, D // 8, BATCH // SUB, 8, SUB), jnp.float32),
    mesh=_mesh,
    compiler_params=pltpu.CompilerParams(
        use_tc_tiling_on_sc=False, needs_layout_passes=False
    ),
    scratch_types=[
        pltpu.VMEM((ROWS_PER_W // SUB, SUB), jnp.int32),
        pltpu.VMEM((2, UNIT, D), jnp.float32),
        pltpu.VMEM((2, 8, KU, 8, SUB + 1), jnp.float32),
        pltpu.VMEM((T, D), jnp.float32),
        pltpu.SemaphoreType.DMA,
        pltpu.SemaphoreType.DMA,
        pltpu.SemaphoreType.DMA,
        pltpu.SemaphoreType.DMA,
    ],
)
def _emb_lookup(x_hbm, tab_hbm, pemb_hbm, out_hbm, idx_v, rows_v, out_v,
                pemb_v, gsem0, gsem1, ssem0, ssem1):
    wid = lax.axis_index("s") * NC + lax.axis_index("c")
    gsems = (gsem0, gsem1)
    ssems = (ssem0, ssem1)
    pltpu.sync_copy(pemb_hbm, pemb_v)
    pltpu.sync_copy(x_hbm.at[wid], idx_v)

    io = lax.iota(jnp.int32, 16)
    gvs = [(16 * c + io) >> 3 for c in range(D // LANES)]
    dvs = [(16 * c + io) & 7 for c in range(D // LANES)]

    def fire_gathers(n, s):
        for j in range(KU):
            pltpu.make_async_copy(
                tab_hbm.at[idx_v.at[KU * n + j]],
                rows_v.at[s, pl.ds(j * SUB, SUB)],
                gsems[s],
            ).start()

    def drain_gathers(n, s):
        for j in range(KU):
            pltpu.make_async_copy(
                tab_hbm.at[idx_v.at[KU * n + j]],
                rows_v.at[s, pl.ds(j * SUB, SUB)],
                gsems[s],
            ).wait()

    def scatter_desc(n, s):
        u = wid * NUNITS + n
        t = u // BLKS
        blk = u % BLKS
        return pltpu.make_async_copy(
            out_v.at[s, :, :, :, pl.ds(0, SUB)],
            out_hbm.at[t, :, pl.ds(blk * KU, KU)],
            ssems[s],
        )

    fire_gathers(0, 0)

    def do_pair(p, carry):
        for s in range(2):
            n = 2 * p + s
            u = wid * NUNITS + n
            t = u // BLKS

            @pl.when(n + 1 < NUNITS)
            def _():
                fire_gathers(n + 1, 1 - s)

            drain_gathers(n, s)

            @pl.when(n >= 2)
            def _():
                scatter_desc(n, s).wait()

            pvs = [pemb_v[t, pl.ds(16 * c, 16)] for c in range(D // LANES)]

            @plsc.parallel_loop(0, UNIT, unroll=2)
            def _(bb, s=s):
                k2 = lax.broadcast(bb >> 7, (16,))
                col = lax.broadcast(bb & 127, (16,))
                for c in range(D // LANES):
                    val = rows_v[s, bb, pl.ds(16 * c, 16)] + pvs[c]
                    plsc.store_scatter(out_v.at[s], [gvs[c], k2, dvs[c], col], val)

            scatter_desc(n, s).start()
        return carry

    lax.fori_loop(0, NUNITS // 2, do_pair, 0)
    scatter_desc(NUNITS - 2, 0).wait()
    scatter_desc(NUNITS - 1, 1).wait()


def kernel(x, t_emb, p_emb):
    xw = x.T.astype(jnp.int32).reshape(NW, ROWS_PER_W // SUB, SUB)
    out5 = _emb_lookup(xw, t_emb, p_emb)
    return jnp.transpose(out5, (2, 4, 0, 1, 3)).reshape(BATCH, T, D)
